# baseline (speedup 1.0000x reference)
import functools

import jax
import jax.numpy as jnp
from jax.experimental import pallas as pl
from jax.experimental.pallas import tpu as pltpu
from jax.experimental.pallas import tpu_sc as plsc

S = 2048
B = 1
D = 768
H = 12
DH = D // H
E = 8
TOPK = 2
DFF = 4 * D
N = S * B
P = N * TOPK
BT = 128
NW = P // BT + E
NSLOT = NW * BT
BM = 512

NSC = 32
GCH = 80
CCH = 32


def _ln_body(x_ref, w_ref, b_ref, o_ref):
    x = x_ref[...]
    m = jnp.mean(x, axis=1, keepdims=True)
    c = x - m
    v = jnp.mean(c * c, axis=1, keepdims=True)
    o_ref[...] = c * jax.lax.rsqrt(v + 1e-5) * w_ref[...] + b_ref[...]


def _ln(x, w, b):
    return pl.pallas_call(
        _ln_body,
        grid=(S // BM,),
        in_specs=[
            pl.BlockSpec((BM, D), lambda i: (i, 0)),
            pl.BlockSpec((1, D), lambda i: (0, 0)),
            pl.BlockSpec((1, D), lambda i: (0, 0)),
        ],
        out_specs=pl.BlockSpec((BM, D), lambda i: (i, 0)),
        out_shape=jax.ShapeDtypeStruct((S, D), jnp.float32),
    )(x, w.reshape(1, D), b.reshape(1, D))


def _mm_body(x_ref, w_ref, b_ref, o_ref):
    o_ref[...] = (
        jnp.dot(x_ref[...], w_ref[...], preferred_element_type=jnp.float32)
        + b_ref[...]
    )


def _mm_res_body(x_ref, w_ref, b_ref, r_ref, o_ref):
    o_ref[...] = (
        jnp.dot(x_ref[...], w_ref[...], preferred_element_type=jnp.float32)
        + b_ref[...]
        + r_ref[...]
    )


def _mm(x, w_t, b, res=None):
    k, n = w_t.shape
    in_specs = [
        pl.BlockSpec((BM, k), lambda i: (i, 0)),
        pl.BlockSpec((k, n), lambda i: (0, 0)),
        pl.BlockSpec((1, n), lambda i: (0, 0)),
    ]
    args = [x, w_t, b.reshape(1, n)]
    body = _mm_body
    if res is not None:
        in_specs.append(pl.BlockSpec((BM, n), lambda i: (i, 0)))
        args.append(res)
        body = _mm_res_body
    return pl.pallas_call(
        body,
        grid=(S // BM,),
        in_specs=in_specs,
        out_specs=pl.BlockSpec((BM, n), lambda i: (i, 0)),
        out_shape=jax.ShapeDtypeStruct((S, n), jnp.float32),
    )(*args)


def _attn_body(q_ref, k_ref, v_ref, o_ref):
    for t in range(2):
        cols = slice(t * DH, (t + 1) * DH)
        q = q_ref[:, cols]
        s = jax.lax.dot_general(
            q, k_ref[:, cols], (((1,), (1,)), ((), ())),
            preferred_element_type=jnp.float32,
            precision=jax.lax.Precision.HIGHEST,
        ) * (1.0 / (DH ** 0.5))
        m = jnp.max(s, axis=1, keepdims=True)
        p = jnp.exp(s - m)
        l = jnp.sum(p, axis=1, keepdims=True)
        o = jax.lax.dot_general(
            p, v_ref[:, cols], (((1,), (0,)), ((), ())),
            preferred_element_type=jnp.float32,
            precision=jax.lax.Precision.HIGHEST,
        )
        o_ref[:, cols] = o / l


def _attn(q_arr, kv_arr, k_off, v_off):
    return pl.pallas_call(
        _attn_body,
        grid=(H // 2,),
        in_specs=[
            pl.BlockSpec((S, 2 * DH), lambda h: (0, h)),
            pl.BlockSpec((S, 2 * DH), lambda h: (0, k_off + h)),
            pl.BlockSpec((S, 2 * DH), lambda h: (0, v_off + h)),
        ],
        out_specs=pl.BlockSpec((S, 2 * DH), lambda h: (0, h)),
        out_shape=jax.ShapeDtypeStruct((S, D), jnp.float32),
        compiler_params=pltpu.CompilerParams(
            vmem_limit_bytes=100 * 1024 * 1024),
    )(q_arr, kv_arr, kv_arr)


def _mha(q_in, kv_in, in_proj_w, in_proj_b, out_proj_w, out_proj_b, res):
    if q_in is kv_in:
        qkv = _mm(q_in, in_proj_w.T, in_proj_b)
        o = _attn(qkv, qkv, H // 2, H)
    else:
        qt = _mm(q_in, in_proj_w[:D].T, in_proj_b[:D])
        kv = _mm(kv_in, in_proj_w[D:].T, in_proj_b[D:])
        o = _attn(qt, kv, 0, H // 2)
    return _mm(o, out_proj_w.T, out_proj_b, res=res)


def _gate_body(x_ref, gw_ref, gb_ref, slot_ref, val_ref, cnt_ref, rcnt_ref):
    logits = (
        jnp.dot(x_ref[...], gw_ref[...], preferred_element_type=jnp.float32)
        + gb_ref[...]
    )
    mx = jnp.max(logits, axis=1, keepdims=True)
    p = jnp.exp(logits - mx)
    p = p / jnp.sum(p, axis=1, keepdims=True)
    cnt_ref[...] = jnp.sum(p, axis=0, keepdims=True)
    lane = jax.lax.broadcasted_iota(jnp.int32, (S, E), 1)
    v1 = jnp.max(p, axis=1, keepdims=True)
    i1 = jnp.min(jnp.where(p == v1, lane, E), axis=1, keepdims=True)
    pm = jnp.where(lane == i1, -jnp.inf, p)
    v2 = jnp.max(pm, axis=1, keepdims=True)
    i2 = jnp.min(jnp.where(pm == v2, lane, E), axis=1, keepdims=True)
    den = v1 + v2
    val_ref[...] = jnp.where(
        lane == 0, v1 / den, jnp.where(lane == 1, v2 / den, 0.0)
    )
    o0 = (lane == i1).astype(jnp.float32)
    o1 = (lane == i2).astype(jnp.float32)
    row_i = jax.lax.broadcasted_iota(jnp.int32, (S, S), 0)
    col_i = jax.lax.broadcasted_iota(jnp.int32, (S, S), 1)
    tri = (col_i < row_i).astype(jnp.float32)
    c0 = jnp.dot(tri, o0, preferred_element_type=jnp.float32)
    c1 = jnp.dot(tri, o1, preferred_element_type=jnp.float32)
    n0 = jnp.sum(o0, axis=0, keepdims=True)
    n1 = jnp.sum(o1, axis=0, keepdims=True)
    rcnt = n0 + n1
    rcnt_ref[...] = rcnt
    rank0 = jnp.sum(c0 * o0, axis=1, keepdims=True)
    rank1 = jnp.sum((c1 + n0) * o1, axis=1, keepdims=True)
    nitems = jnp.floor((rcnt + (BT - 1)) * (1.0 / BT))
    lr = jax.lax.broadcasted_iota(jnp.int32, (E, E), 0)
    lc = jax.lax.broadcasted_iota(jnp.int32, (E, E), 1)
    tri8 = (lr < lc).astype(jnp.float32)
    ibase = jnp.dot(nitems, tri8, preferred_element_type=jnp.float32)
    ib0 = jnp.sum(ibase * o0, axis=1, keepdims=True)
    ib1 = jnp.sum(ibase * o1, axis=1, keepdims=True)
    blk0 = jnp.floor(rank0 * (1.0 / BT))
    blk1 = jnp.floor(rank1 * (1.0 / BT))
    slot0 = (ib0 + blk0) * BT + (rank0 - blk0 * BT)
    slot1 = (ib1 + blk1) * BT + (rank1 - blk1 * BT)
    slot_ref[...] = jnp.where(
        lane == 0, slot0.astype(jnp.int32),
        jnp.where(lane == 1, slot1.astype(jnp.int32), 0))


def _gate(x, gw_t, gb):
    full = pl.BlockSpec((S, E), lambda: (0, 0))
    one = pl.BlockSpec((1, E), lambda: (0, 0))
    slot, val, cnt, rcnt = pl.pallas_call(
        _gate_body,
        in_specs=[
            pl.BlockSpec((S, D), lambda: (0, 0)),
            pl.BlockSpec((D, E), lambda: (0, 0)),
            one,
        ],
        out_specs=[full, full, one, one],
        out_shape=[
            jax.ShapeDtypeStruct((S, E), jnp.int32),
            jax.ShapeDtypeStruct((S, E), jnp.float32),
            jax.ShapeDtypeStruct((1, E), jnp.float32),
            jax.ShapeDtypeStruct((1, E), jnp.float32),
        ],
        compiler_params=pltpu.CompilerParams(
            vmem_limit_bytes=100 * 1024 * 1024),
    )(x, gw_t, gb.reshape(1, E))
    return slot, val, cnt[0], rcnt[0]


def _ffn_body(eidx_ref, xs_ref, w1_ref, b1_ref, w2_ref, b2_ref, o_ref):
    h = jnp.maximum(
        jax.lax.dot_general(
            xs_ref[...], w1_ref[0], (((1,), (1,)), ((), ())),
            preferred_element_type=jnp.float32,
        ) + b1_ref[0],
        0.0,
    )
    o_ref[...] = jax.lax.dot_general(
        h, w2_ref[0], (((1,), (1,)), ((), ())),
        preferred_element_type=jnp.float32,
    ) + b2_ref[0]


def _moe_ffn(xs, expert_of_item, w1, b1, w2, b2):
    grid_spec = pltpu.PrefetchScalarGridSpec(
        num_scalar_prefetch=1,
        grid=(NW,),
        in_specs=[
            pl.BlockSpec((BT, D), lambda w, e: (w, 0)),
            pl.BlockSpec((1, DFF, D), lambda w, e: (e[w], 0, 0)),
            pl.BlockSpec((1, 1, DFF), lambda w, e: (e[w], 0, 0)),
            pl.BlockSpec((1, D, DFF), lambda w, e: (e[w], 0, 0)),
            pl.BlockSpec((1, 1, D), lambda w, e: (e[w], 0, 0)),
        ],
        out_specs=pl.BlockSpec((BT, D), lambda w, e: (w, 0)),
    )
    return pl.pallas_call(
        _ffn_body,
        grid_spec=grid_spec,
        out_shape=jax.ShapeDtypeStruct((NSLOT, D), jnp.float32),
        compiler_params=pltpu.CompilerParams(
            vmem_limit_bytes=100 * 1024 * 1024),
    )(expert_of_item, xs,
      w1, b1.reshape(E, 1, DFF), w2, b2.reshape(E, 1, D))


def _sc_mesh():
    return plsc.VectorSubcoreMesh(
        core_axis_name="c", subcore_axis_name="s",
        num_cores=2, num_subcores=16)


PCH = 64
PPW = P // NSC


def _sc_permute(table, slots3):
    @functools.partial(
        pl.kernel,
        out_type=jax.ShapeDtypeStruct((NSLOT, D), jnp.float32),
        mesh=_sc_mesh(),
        scratch_types=[
            pltpu.VMEM((PCH,), jnp.int32),
            pltpu.VMEM((PCH, D), jnp.float32),
            pltpu.SemaphoreType.DMA,
        ],
    )
    def k(table_hbm, slots_hbm, out_hbm, idx_v, rows_v, sem):
        wid = jax.lax.axis_index("s") * 2 + jax.lax.axis_index("c")
        for c in range(PPW // PCH):
            t0 = jax.lax.rem(wid * PPW + c * PCH, S)
            pltpu.sync_copy(table_hbm.at[pl.ds(t0, PCH)], rows_v)
            pltpu.sync_copy(slots_hbm.at[wid, c], idx_v)
            pltpu.async_copy(rows_v, out_hbm.at[idx_v], sem).wait()

    return k(table, slots3)


def _sc_combine(ys, p0, p1, g0, g1, res):
    tok_per_w = S // NSC

    @functools.partial(
        pl.kernel,
        out_type=jax.ShapeDtypeStruct((S, D), jnp.float32),
        mesh=_sc_mesh(),
        scratch_types=[
            pltpu.VMEM((CCH,), jnp.int32),
            pltpu.VMEM((CCH, 16), jnp.float32),
            pltpu.VMEM((CCH, 16), jnp.float32),
            pltpu.VMEM((CCH, D), jnp.float32),
            pltpu.VMEM((CCH, D), jnp.float32),
            pltpu.VMEM((CCH, D), jnp.float32),
            pltpu.SemaphoreType.DMA,
        ],
    )
    def k(ys_hbm, p0_hbm, p1_hbm, g0_hbm, g1_hbm, res_hbm, out_hbm,
          idx_v, g0_v, g1_v, a_v, b_v, r_v, sem):
        wid = jax.lax.axis_index("s") * 2 + jax.lax.axis_index("c")
        base = wid * tok_per_w
        for c in range(tok_per_w // CCH):
            off = base + c * CCH
            pltpu.sync_copy(res_hbm.at[pl.ds(off, CCH)], r_v)
            pltpu.sync_copy(g0_hbm.at[pl.ds(off, CCH)], g0_v)
            pltpu.sync_copy(g1_hbm.at[pl.ds(off, CCH)], g1_v)
            pltpu.sync_copy(p0_hbm.at[pl.ds(off, CCH)], idx_v)
            pltpu.async_copy(ys_hbm.at[idx_v], a_v, sem).wait()
            pltpu.sync_copy(p1_hbm.at[pl.ds(off, CCH)], idx_v)
            pltpu.async_copy(ys_hbm.at[idx_v], b_v, sem).wait()

            def row(rr, carry):
                ga = g0_v[rr, :]
                gb = g1_v[rr, :]
                for cc in range(D // 16):
                    sl = pl.ds(cc * 16, 16)
                    r_v[rr, sl] = (r_v[rr, sl] + ga * a_v[rr, sl]
                                   + gb * b_v[rr, sl])
                return carry

            jax.lax.fori_loop(0, CCH, row, 0)
            pltpu.sync_copy(r_v, out_hbm.at[pl.ds(off, CCH)])

    return k(ys, p0, p1, g0, g1, res)


def _moe(x, gw_t, gb, w1, b1, w2, b2, res):
    slot, val, cnt, rcnt = _gate(x, gw_t, gb)
    p0 = slot[:, 0]
    p1 = slot[:, 1]
    g0 = jnp.broadcast_to(val[:, 0:1], (S, 16))
    g1 = jnp.broadcast_to(val[:, 1:2], (S, 16))
    slots3 = jnp.concatenate([p0, p1]).reshape(NSC, PPW // PCH, PCH)
    nitems = jnp.floor((rcnt + (BT - 1)) * (1.0 / BT))
    cum = jnp.cumsum(nitems)
    e_w = jnp.sum((jnp.arange(NW)[:, None] >= cum[None, :]).astype(
        jnp.int32), axis=1)
    e_w = jnp.clip(e_w, 0, E - 1).astype(jnp.int32)
    xs = _sc_permute(x, slots3)
    ys = _moe_ffn(xs, e_w, w1, b1, w2, b2)
    out = _sc_combine(ys, p0, p1, g0, g1, res)
    aux = E * jnp.sum((cnt / jnp.sum(cnt)) * (cnt / N))
    return out, aux


def kernel(x_context, x_target, in_proj_w, in_proj_b, out_proj_w,
           out_proj_b, gate_w, gate_b, w1, b1, w2, b2, ln_c1_w, ln_c1_b,
           ln_c2_w, ln_c2_b, ln_t1_w, ln_t1_b, ln_t2_w, ln_t2_b):
    xc0 = x_context.reshape(S, D)
    xt0 = x_target.reshape(S, D)
    gw_t = gate_w.T

    xcn = _ln(xc0, ln_c1_w, ln_c1_b)
    xc1 = _mha(xcn, xcn, in_proj_w, in_proj_b, out_proj_w, out_proj_b, xc0)
    xcn2 = _ln(xc1, ln_c2_w, ln_c2_b)
    xc2, aux1 = _moe(xcn2, gw_t, gate_b, w1, b1, w2, b2, xc1)

    xtn = _ln(xt0, ln_t1_w, ln_t1_b)
    xt1 = _mha(xtn, xc2, in_proj_w, in_proj_b, out_proj_w, out_proj_b, xt0)
    xtn2 = _ln(xt1, ln_t2_w, ln_t2_b)
    xt2, aux2 = _moe(xtn2, gw_t, gate_b, w1, b1, w2, b2, xt1)

    aux = 0.01 * (aux1 + aux2)
    return xc2.reshape(S, B, D), xt2.reshape(S, B, D), aux


# speedup vs baseline: 1.9666x; 1.9666x over previous
import functools

import jax
import jax.numpy as jnp
from jax.experimental import pallas as pl
from jax.experimental.pallas import tpu as pltpu
from jax.experimental.pallas import tpu_sc as plsc

S = 2048
B = 1
D = 768
H = 12
DH = D // H
E = 8
TOPK = 2
DFF = 4 * D
N = S * B
P = N * TOPK
BT = 128
NW = P // BT + E
NSLOT = NW * BT
BM = 512

NSC = 32
GCH = 80
CCH = 32


def _ln_body(x_ref, w_ref, b_ref, o_ref):
    x = x_ref[...]
    m = jnp.mean(x, axis=1, keepdims=True)
    c = x - m
    v = jnp.mean(c * c, axis=1, keepdims=True)
    o_ref[...] = c * jax.lax.rsqrt(v + 1e-5) * w_ref[...] + b_ref[...]


def _ln(x, w, b):
    return pl.pallas_call(
        _ln_body,
        grid=(S // BM,),
        in_specs=[
            pl.BlockSpec((BM, D), lambda i: (i, 0)),
            pl.BlockSpec((1, D), lambda i: (0, 0)),
            pl.BlockSpec((1, D), lambda i: (0, 0)),
        ],
        out_specs=pl.BlockSpec((BM, D), lambda i: (i, 0)),
        out_shape=jax.ShapeDtypeStruct((S, D), jnp.float32),
    )(x, w.reshape(1, D), b.reshape(1, D))


def _mm_body(x_ref, w_ref, b_ref, o_ref):
    o_ref[...] = (
        jnp.dot(x_ref[...], w_ref[...], preferred_element_type=jnp.float32)
        + b_ref[...]
    )


def _mm_res_body(x_ref, w_ref, b_ref, r_ref, o_ref):
    o_ref[...] = (
        jnp.dot(x_ref[...], w_ref[...], preferred_element_type=jnp.float32)
        + b_ref[...]
        + r_ref[...]
    )


def _mm(x, w_t, b, res=None):
    k, n = w_t.shape
    in_specs = [
        pl.BlockSpec((BM, k), lambda i: (i, 0)),
        pl.BlockSpec((k, n), lambda i: (0, 0)),
        pl.BlockSpec((1, n), lambda i: (0, 0)),
    ]
    args = [x, w_t, b.reshape(1, n)]
    body = _mm_body
    if res is not None:
        in_specs.append(pl.BlockSpec((BM, n), lambda i: (i, 0)))
        args.append(res)
        body = _mm_res_body
    return pl.pallas_call(
        body,
        grid=(S // BM,),
        in_specs=in_specs,
        out_specs=pl.BlockSpec((BM, n), lambda i: (i, 0)),
        out_shape=jax.ShapeDtypeStruct((S, n), jnp.float32),
    )(*args)


def _attn_body(q_ref, k_ref, v_ref, o_ref):
    for t in range(2):
        cols = slice(t * DH, (t + 1) * DH)
        q = q_ref[:, cols]
        k = k_ref[:, cols]
        qh = q.astype(jnp.bfloat16).astype(jnp.float32)
        ql = q - qh
        kh = k.astype(jnp.bfloat16).astype(jnp.float32)
        kl = k - kh
        dn = (((1,), (1,)), ((), ()))
        s = (
            jax.lax.dot_general(qh, kh, dn,
                                preferred_element_type=jnp.float32)
            + jax.lax.dot_general(qh, kl, dn,
                                  preferred_element_type=jnp.float32)
            + jax.lax.dot_general(ql, kh, dn,
                                  preferred_element_type=jnp.float32)
        ) * (1.0 / (DH ** 0.5))
        m = jnp.max(s, axis=1, keepdims=True)
        p = jnp.exp(s - m)
        l = jnp.sum(p, axis=1, keepdims=True)
        o = jax.lax.dot_general(
            p, v_ref[:, cols], (((1,), (0,)), ((), ())),
            preferred_element_type=jnp.float32,
        )
        o_ref[:, cols] = o / l


def _attn(q_arr, kv_arr, k_off, v_off):
    return pl.pallas_call(
        _attn_body,
        grid=(H // 2,),
        in_specs=[
            pl.BlockSpec((S, 2 * DH), lambda h: (0, h)),
            pl.BlockSpec((S, 2 * DH), lambda h: (0, k_off + h)),
            pl.BlockSpec((S, 2 * DH), lambda h: (0, v_off + h)),
        ],
        out_specs=pl.BlockSpec((S, 2 * DH), lambda h: (0, h)),
        out_shape=jax.ShapeDtypeStruct((S, D), jnp.float32),
        compiler_params=pltpu.CompilerParams(
            vmem_limit_bytes=100 * 1024 * 1024),
    )(q_arr, kv_arr, kv_arr)


def _mha(q_in, kv_in, in_proj_w, in_proj_b, out_proj_w, out_proj_b, res):
    if q_in is kv_in:
        qkv = _mm(q_in, in_proj_w.T, in_proj_b)
        o = _attn(qkv, qkv, H // 2, H)
    else:
        qt = _mm(q_in, in_proj_w[:D].T, in_proj_b[:D])
        kv = _mm(kv_in, in_proj_w[D:].T, in_proj_b[D:])
        o = _attn(qt, kv, 0, H // 2)
    return _mm(o, out_proj_w.T, out_proj_b, res=res)


def _gate_body(x_ref, gw_ref, gb_ref, slot_ref, val_ref, cnt_ref, rcnt_ref):
    logits = (
        jnp.dot(x_ref[...], gw_ref[...], preferred_element_type=jnp.float32)
        + gb_ref[...]
    )
    mx = jnp.max(logits, axis=1, keepdims=True)
    p = jnp.exp(logits - mx)
    p = p / jnp.sum(p, axis=1, keepdims=True)
    cnt_ref[...] = jnp.sum(p, axis=0, keepdims=True)
    lane = jax.lax.broadcasted_iota(jnp.int32, (S, E), 1)
    v1 = jnp.max(p, axis=1, keepdims=True)
    i1 = jnp.min(jnp.where(p == v1, lane, E), axis=1, keepdims=True)
    pm = jnp.where(lane == i1, -jnp.inf, p)
    v2 = jnp.max(pm, axis=1, keepdims=True)
    i2 = jnp.min(jnp.where(pm == v2, lane, E), axis=1, keepdims=True)
    den = v1 + v2
    val_ref[...] = jnp.where(
        lane == 0, v1 / den, jnp.where(lane == 1, v2 / den, 0.0)
    )
    o0 = (lane == i1).astype(jnp.float32)
    o1 = (lane == i2).astype(jnp.float32)
    row_i = jax.lax.broadcasted_iota(jnp.int32, (S, S), 0)
    col_i = jax.lax.broadcasted_iota(jnp.int32, (S, S), 1)
    tri = (col_i < row_i).astype(jnp.float32)
    c0 = jnp.dot(tri, o0, preferred_element_type=jnp.float32)
    c1 = jnp.dot(tri, o1, preferred_element_type=jnp.float32)
    n0 = jnp.sum(o0, axis=0, keepdims=True)
    n1 = jnp.sum(o1, axis=0, keepdims=True)
    rcnt = n0 + n1
    rcnt_ref[...] = rcnt
    rank0 = jnp.sum(c0 * o0, axis=1, keepdims=True)
    rank1 = jnp.sum((c1 + n0) * o1, axis=1, keepdims=True)
    nitems = jnp.floor((rcnt + (BT - 1)) * (1.0 / BT))
    lr = jax.lax.broadcasted_iota(jnp.int32, (E, E), 0)
    lc = jax.lax.broadcasted_iota(jnp.int32, (E, E), 1)
    tri8 = (lr < lc).astype(jnp.float32)
    ibase = jnp.dot(nitems, tri8, preferred_element_type=jnp.float32)
    ib0 = jnp.sum(ibase * o0, axis=1, keepdims=True)
    ib1 = jnp.sum(ibase * o1, axis=1, keepdims=True)
    blk0 = jnp.floor(rank0 * (1.0 / BT))
    blk1 = jnp.floor(rank1 * (1.0 / BT))
    slot0 = (ib0 + blk0) * BT + (rank0 - blk0 * BT)
    slot1 = (ib1 + blk1) * BT + (rank1 - blk1 * BT)
    slot_ref[...] = jnp.where(
        lane == 0, slot0.astype(jnp.int32),
        jnp.where(lane == 1, slot1.astype(jnp.int32), 0))


def _gate(x, gw_t, gb):
    full = pl.BlockSpec((S, E), lambda: (0, 0))
    one = pl.BlockSpec((1, E), lambda: (0, 0))
    slot, val, cnt, rcnt = pl.pallas_call(
        _gate_body,
        in_specs=[
            pl.BlockSpec((S, D), lambda: (0, 0)),
            pl.BlockSpec((D, E), lambda: (0, 0)),
            one,
        ],
        out_specs=[full, full, one, one],
        out_shape=[
            jax.ShapeDtypeStruct((S, E), jnp.int32),
            jax.ShapeDtypeStruct((S, E), jnp.float32),
            jax.ShapeDtypeStruct((1, E), jnp.float32),
            jax.ShapeDtypeStruct((1, E), jnp.float32),
        ],
        compiler_params=pltpu.CompilerParams(
            vmem_limit_bytes=100 * 1024 * 1024),
    )(x, gw_t, gb.reshape(1, E))
    return slot, val, cnt[0], rcnt[0]


def _ffn_body(eidx_ref, xs_ref, w1_ref, b1_ref, w2_ref, b2_ref, o_ref):
    h = jnp.maximum(
        jax.lax.dot_general(
            xs_ref[...], w1_ref[0], (((1,), (1,)), ((), ())),
            preferred_element_type=jnp.float32,
        ) + b1_ref[0],
        0.0,
    )
    o_ref[...] = jax.lax.dot_general(
        h, w2_ref[0], (((1,), (1,)), ((), ())),
        preferred_element_type=jnp.float32,
    ) + b2_ref[0]


def _moe_ffn(xs, expert_of_item, w1, b1, w2, b2):
    grid_spec = pltpu.PrefetchScalarGridSpec(
        num_scalar_prefetch=1,
        grid=(NW,),
        in_specs=[
            pl.BlockSpec((BT, D), lambda w, e: (w, 0)),
            pl.BlockSpec((1, DFF, D), lambda w, e: (e[w], 0, 0)),
            pl.BlockSpec((1, 1, DFF), lambda w, e: (e[w], 0, 0)),
            pl.BlockSpec((1, D, DFF), lambda w, e: (e[w], 0, 0)),
            pl.BlockSpec((1, 1, D), lambda w, e: (e[w], 0, 0)),
        ],
        out_specs=pl.BlockSpec((BT, D), lambda w, e: (w, 0)),
    )
    return pl.pallas_call(
        _ffn_body,
        grid_spec=grid_spec,
        out_shape=jax.ShapeDtypeStruct((NSLOT, D), jnp.float32),
        compiler_params=pltpu.CompilerParams(
            vmem_limit_bytes=100 * 1024 * 1024),
    )(expert_of_item, xs,
      w1, b1.reshape(E, 1, DFF), w2, b2.reshape(E, 1, D))


def _sc_mesh():
    return plsc.VectorSubcoreMesh(
        core_axis_name="c", subcore_axis_name="s",
        num_cores=2, num_subcores=16)


PCH = 64
PPW = P // NSC


def _sc_permute(table, slots3):
    @functools.partial(
        pl.kernel,
        out_type=jax.ShapeDtypeStruct((NSLOT, D), jnp.float32),
        mesh=_sc_mesh(),
        scratch_types=[
            pltpu.VMEM((PCH,), jnp.int32),
            pltpu.VMEM((PCH, D), jnp.float32),
            pltpu.SemaphoreType.DMA,
        ],
    )
    def k(table_hbm, slots_hbm, out_hbm, idx_v, rows_v, sem):
        wid = jax.lax.axis_index("s") * 2 + jax.lax.axis_index("c")
        for c in range(PPW // PCH):
            t0 = jax.lax.rem(wid * PPW + c * PCH, S)
            pltpu.sync_copy(table_hbm.at[pl.ds(t0, PCH)], rows_v)
            pltpu.sync_copy(slots_hbm.at[wid, c], idx_v)
            pltpu.async_copy(rows_v, out_hbm.at[idx_v], sem).wait()

    return k(table, slots3)


def _sc_combine(ys, p0, p1, g0, g1, res):
    tok_per_w = S // NSC

    @functools.partial(
        pl.kernel,
        out_type=jax.ShapeDtypeStruct((S, D), jnp.float32),
        mesh=_sc_mesh(),
        scratch_types=[
            pltpu.VMEM((CCH,), jnp.int32),
            pltpu.VMEM((CCH, 16), jnp.float32),
            pltpu.VMEM((CCH, 16), jnp.float32),
            pltpu.VMEM((CCH, D), jnp.float32),
            pltpu.VMEM((CCH, D), jnp.float32),
            pltpu.VMEM((CCH, D), jnp.float32),
            pltpu.SemaphoreType.DMA,
        ],
    )
    def k(ys_hbm, p0_hbm, p1_hbm, g0_hbm, g1_hbm, res_hbm, out_hbm,
          idx_v, g0_v, g1_v, a_v, b_v, r_v, sem):
        wid = jax.lax.axis_index("s") * 2 + jax.lax.axis_index("c")
        base = wid * tok_per_w
        for c in range(tok_per_w // CCH):
            off = base + c * CCH
            pltpu.sync_copy(res_hbm.at[pl.ds(off, CCH)], r_v)
            pltpu.sync_copy(g0_hbm.at[pl.ds(off, CCH)], g0_v)
            pltpu.sync_copy(g1_hbm.at[pl.ds(off, CCH)], g1_v)
            pltpu.sync_copy(p0_hbm.at[pl.ds(off, CCH)], idx_v)
            pltpu.async_copy(ys_hbm.at[idx_v], a_v, sem).wait()
            pltpu.sync_copy(p1_hbm.at[pl.ds(off, CCH)], idx_v)
            pltpu.async_copy(ys_hbm.at[idx_v], b_v, sem).wait()

            def row(rr, carry):
                ga = g0_v[rr, :]
                gb = g1_v[rr, :]
                for cc in range(D // 16):
                    sl = pl.ds(cc * 16, 16)
                    r_v[rr, sl] = (r_v[rr, sl] + ga * a_v[rr, sl]
                                   + gb * b_v[rr, sl])
                return carry

            jax.lax.fori_loop(0, CCH, row, 0)
            pltpu.sync_copy(r_v, out_hbm.at[pl.ds(off, CCH)])

    return k(ys, p0, p1, g0, g1, res)


def _moe(x, gw_t, gb, w1, b1, w2, b2, res):
    slot, val, cnt, rcnt = _gate(x, gw_t, gb)
    p0 = slot[:, 0]
    p1 = slot[:, 1]
    g0 = jnp.broadcast_to(val[:, 0:1], (S, 16))
    g1 = jnp.broadcast_to(val[:, 1:2], (S, 16))
    slots3 = jnp.concatenate([p0, p1]).reshape(NSC, PPW // PCH, PCH)
    nitems = jnp.floor((rcnt + (BT - 1)) * (1.0 / BT))
    cum = jnp.cumsum(nitems)
    e_w = jnp.sum((jnp.arange(NW)[:, None] >= cum[None, :]).astype(
        jnp.int32), axis=1)
    e_w = jnp.clip(e_w, 0, E - 1).astype(jnp.int32)
    xs = _sc_permute(x, slots3)
    ys = _moe_ffn(xs, e_w, w1, b1, w2, b2)
    out = _sc_combine(ys, p0, p1, g0, g1, res)
    aux = E * jnp.sum((cnt / jnp.sum(cnt)) * (cnt / N))
    return out, aux


def kernel(x_context, x_target, in_proj_w, in_proj_b, out_proj_w,
           out_proj_b, gate_w, gate_b, w1, b1, w2, b2, ln_c1_w, ln_c1_b,
           ln_c2_w, ln_c2_b, ln_t1_w, ln_t1_b, ln_t2_w, ln_t2_b):
    xc0 = x_context.reshape(S, D)
    xt0 = x_target.reshape(S, D)
    gw_t = gate_w.T

    xcn = _ln(xc0, ln_c1_w, ln_c1_b)
    xc1 = _mha(xcn, xcn, in_proj_w, in_proj_b, out_proj_w, out_proj_b, xc0)
    xcn2 = _ln(xc1, ln_c2_w, ln_c2_b)
    xc2, aux1 = _moe(xcn2, gw_t, gate_b, w1, b1, w2, b2, xc1)

    xtn = _ln(xt0, ln_t1_w, ln_t1_b)
    xt1 = _mha(xtn, xc2, in_proj_w, in_proj_b, out_proj_w, out_proj_b, xt0)
    xtn2 = _ln(xt1, ln_t2_w, ln_t2_b)
    xt2, aux2 = _moe(xtn2, gw_t, gate_b, w1, b1, w2, b2, xt1)

    aux = 0.01 * (aux1 + aux2)
    return xc2.reshape(S, B, D), xt2.reshape(S, B, D), aux


# speedup vs baseline: 2.2540x; 1.1462x over previous
import functools

import jax
import jax.numpy as jnp
from jax.experimental import pallas as pl
from jax.experimental.pallas import tpu as pltpu
from jax.experimental.pallas import tpu_sc as plsc

S = 2048
B = 1
D = 768
H = 12
DH = D // H
E = 8
TOPK = 2
DFF = 4 * D
N = S * B
P = N * TOPK
BT = 128
NW = P // BT + E
NSLOT = NW * BT
BM = 512

NSC = 32
GCH = 80
CCH = 32


def _ln_body(x_ref, w_ref, b_ref, o_ref):
    x = x_ref[...]
    m = jnp.mean(x, axis=1, keepdims=True)
    c = x - m
    v = jnp.mean(c * c, axis=1, keepdims=True)
    o_ref[...] = c * jax.lax.rsqrt(v + 1e-5) * w_ref[...] + b_ref[...]


def _ln(x, w, b):
    return pl.pallas_call(
        _ln_body,
        grid=(S // BM,),
        in_specs=[
            pl.BlockSpec((BM, D), lambda i: (i, 0)),
            pl.BlockSpec((1, D), lambda i: (0, 0)),
            pl.BlockSpec((1, D), lambda i: (0, 0)),
        ],
        out_specs=pl.BlockSpec((BM, D), lambda i: (i, 0)),
        out_shape=jax.ShapeDtypeStruct((S, D), jnp.float32),
    )(x, w.reshape(1, D), b.reshape(1, D))


def _mm_body(x_ref, w_ref, b_ref, o_ref):
    o_ref[...] = (
        jnp.dot(x_ref[...], w_ref[...], preferred_element_type=jnp.float32)
        + b_ref[...]
    )


def _mm_res_body(x_ref, w_ref, b_ref, r_ref, o_ref):
    o_ref[...] = (
        jnp.dot(x_ref[...], w_ref[...], preferred_element_type=jnp.float32)
        + b_ref[...]
        + r_ref[...]
    )


def _mm(x, w_t, b, res=None):
    k, n = w_t.shape
    in_specs = [
        pl.BlockSpec((BM, k), lambda i: (i, 0)),
        pl.BlockSpec((k, n), lambda i: (0, 0)),
        pl.BlockSpec((1, n), lambda i: (0, 0)),
    ]
    args = [x, w_t, b.reshape(1, n)]
    body = _mm_body
    if res is not None:
        in_specs.append(pl.BlockSpec((BM, n), lambda i: (i, 0)))
        args.append(res)
        body = _mm_res_body
    return pl.pallas_call(
        body,
        grid=(S // BM,),
        in_specs=in_specs,
        out_specs=pl.BlockSpec((BM, n), lambda i: (i, 0)),
        out_shape=jax.ShapeDtypeStruct((S, n), jnp.float32),
    )(*args)


def _attn_body(q_ref, k_ref, v_ref, o_ref):
    lane = jax.lax.broadcasted_iota(jnp.int32, (S, 2 * DH), 1)
    qf = q_ref[...]
    kf = k_ref[...]
    vf = v_ref[...]
    acc = None
    for t in range(2):
        half = (lane >= t * DH) & (lane < (t + 1) * DH)
        qm = jnp.where(half, qf, 0.0)
        km = jnp.where(half, kf, 0.0)
        vm = jnp.where(half, vf, 0.0)
        s = jax.lax.dot_general(
            qm, km, (((1,), (1,)), ((), ())),
            preferred_element_type=jnp.float32,
        ) * (1.0 / (DH ** 0.5))
        m = jnp.max(s, axis=1, keepdims=True)
        p = jnp.exp(s - m)
        l = jnp.sum(p, axis=1, keepdims=True)
        o = jax.lax.dot_general(
            p, vm, (((1,), (0,)), ((), ())),
            preferred_element_type=jnp.float32,
        ) / l
        acc = o if acc is None else acc + o
    o_ref[...] = acc


def _attn(q_arr, kv_arr, k_off, v_off):
    return pl.pallas_call(
        _attn_body,
        grid=(H // 2,),
        in_specs=[
            pl.BlockSpec((S, 2 * DH), lambda h: (0, h)),
            pl.BlockSpec((S, 2 * DH), lambda h: (0, k_off + h)),
            pl.BlockSpec((S, 2 * DH), lambda h: (0, v_off + h)),
        ],
        out_specs=pl.BlockSpec((S, 2 * DH), lambda h: (0, h)),
        out_shape=jax.ShapeDtypeStruct((S, D), jnp.float32),
        compiler_params=pltpu.CompilerParams(
            vmem_limit_bytes=100 * 1024 * 1024),
    )(q_arr, kv_arr, kv_arr)


def _mha(q_in, kv_in, in_proj_w, in_proj_b, out_proj_w, out_proj_b, res):
    if q_in is kv_in:
        qkv = _mm(q_in, in_proj_w.T, in_proj_b)
        o = _attn(qkv, qkv, H // 2, H)
    else:
        qt = _mm(q_in, in_proj_w[:D].T, in_proj_b[:D])
        kv = _mm(kv_in, in_proj_w[D:].T, in_proj_b[D:])
        o = _attn(qt, kv, 0, H // 2)
    return _mm(o, out_proj_w.T, out_proj_b, res=res)


def _gate_body(x_ref, gw_ref, gb_ref, slot_ref, val_ref, cnt_ref, rcnt_ref):
    logits = (
        jnp.dot(x_ref[...], gw_ref[...], preferred_element_type=jnp.float32)
        + gb_ref[...]
    )
    mx = jnp.max(logits, axis=1, keepdims=True)
    p = jnp.exp(logits - mx)
    p = p / jnp.sum(p, axis=1, keepdims=True)
    cnt_ref[...] = jnp.sum(p, axis=0, keepdims=True)
    lane = jax.lax.broadcasted_iota(jnp.int32, (S, E), 1)
    v1 = jnp.max(p, axis=1, keepdims=True)
    i1 = jnp.min(jnp.where(p == v1, lane, E), axis=1, keepdims=True)
    pm = jnp.where(lane == i1, -jnp.inf, p)
    v2 = jnp.max(pm, axis=1, keepdims=True)
    i2 = jnp.min(jnp.where(pm == v2, lane, E), axis=1, keepdims=True)
    den = v1 + v2
    val_ref[...] = jnp.where(
        lane == 0, v1 / den, jnp.where(lane == 1, v2 / den, 0.0)
    )
    o0 = (lane == i1).astype(jnp.float32)
    o1 = (lane == i2).astype(jnp.float32)
    row_i = jax.lax.broadcasted_iota(jnp.int32, (S, S), 0)
    col_i = jax.lax.broadcasted_iota(jnp.int32, (S, S), 1)
    tri = (col_i < row_i).astype(jnp.float32)
    c0 = jnp.dot(tri, o0, preferred_element_type=jnp.float32)
    c1 = jnp.dot(tri, o1, preferred_element_type=jnp.float32)
    n0 = jnp.sum(o0, axis=0, keepdims=True)
    n1 = jnp.sum(o1, axis=0, keepdims=True)
    rcnt = n0 + n1
    rcnt_ref[...] = rcnt
    rank0 = jnp.sum(c0 * o0, axis=1, keepdims=True)
    rank1 = jnp.sum((c1 + n0) * o1, axis=1, keepdims=True)
    nitems = jnp.floor((rcnt + (BT - 1)) * (1.0 / BT))
    lr = jax.lax.broadcasted_iota(jnp.int32, (E, E), 0)
    lc = jax.lax.broadcasted_iota(jnp.int32, (E, E), 1)
    tri8 = (lr < lc).astype(jnp.float32)
    ibase = jnp.dot(nitems, tri8, preferred_element_type=jnp.float32)
    ib0 = jnp.sum(ibase * o0, axis=1, keepdims=True)
    ib1 = jnp.sum(ibase * o1, axis=1, keepdims=True)
    blk0 = jnp.floor(rank0 * (1.0 / BT))
    blk1 = jnp.floor(rank1 * (1.0 / BT))
    slot0 = (ib0 + blk0) * BT + (rank0 - blk0 * BT)
    slot1 = (ib1 + blk1) * BT + (rank1 - blk1 * BT)
    slot_ref[...] = jnp.where(
        lane == 0, slot0.astype(jnp.int32),
        jnp.where(lane == 1, slot1.astype(jnp.int32), 0))


def _gate(x, gw_t, gb):
    full = pl.BlockSpec((S, E), lambda: (0, 0))
    one = pl.BlockSpec((1, E), lambda: (0, 0))
    slot, val, cnt, rcnt = pl.pallas_call(
        _gate_body,
        in_specs=[
            pl.BlockSpec((S, D), lambda: (0, 0)),
            pl.BlockSpec((D, E), lambda: (0, 0)),
            one,
        ],
        out_specs=[full, full, one, one],
        out_shape=[
            jax.ShapeDtypeStruct((S, E), jnp.int32),
            jax.ShapeDtypeStruct((S, E), jnp.float32),
            jax.ShapeDtypeStruct((1, E), jnp.float32),
            jax.ShapeDtypeStruct((1, E), jnp.float32),
        ],
        compiler_params=pltpu.CompilerParams(
            vmem_limit_bytes=100 * 1024 * 1024),
    )(x, gw_t, gb.reshape(1, E))
    return slot, val, cnt[0], rcnt[0]


def _ffn_body(eidx_ref, xs_ref, w1_ref, b1_ref, w2_ref, b2_ref, o_ref):
    h = jnp.maximum(
        jax.lax.dot_general(
            xs_ref[...], w1_ref[0], (((1,), (1,)), ((), ())),
            preferred_element_type=jnp.float32,
        ) + b1_ref[0],
        0.0,
    )
    o_ref[...] = jax.lax.dot_general(
        h, w2_ref[0], (((1,), (1,)), ((), ())),
        preferred_element_type=jnp.float32,
    ) + b2_ref[0]


def _moe_ffn(xs, expert_of_item, w1, b1, w2, b2):
    grid_spec = pltpu.PrefetchScalarGridSpec(
        num_scalar_prefetch=1,
        grid=(NW,),
        in_specs=[
            pl.BlockSpec((BT, D), lambda w, e: (w, 0)),
            pl.BlockSpec((1, DFF, D), lambda w, e: (e[w], 0, 0)),
            pl.BlockSpec((1, 1, DFF), lambda w, e: (e[w], 0, 0)),
            pl.BlockSpec((1, D, DFF), lambda w, e: (e[w], 0, 0)),
            pl.BlockSpec((1, 1, D), lambda w, e: (e[w], 0, 0)),
        ],
        out_specs=pl.BlockSpec((BT, D), lambda w, e: (w, 0)),
    )
    return pl.pallas_call(
        _ffn_body,
        grid_spec=grid_spec,
        out_shape=jax.ShapeDtypeStruct((NSLOT, D), jnp.float32),
        compiler_params=pltpu.CompilerParams(
            vmem_limit_bytes=100 * 1024 * 1024),
    )(expert_of_item, xs,
      w1, b1.reshape(E, 1, DFF), w2, b2.reshape(E, 1, D))


def _sc_mesh():
    return plsc.VectorSubcoreMesh(
        core_axis_name="c", subcore_axis_name="s",
        num_cores=2, num_subcores=16)


PCH = 64
PPW = P // NSC


def _sc_permute(table, slots3):
    @functools.partial(
        pl.kernel,
        out_type=jax.ShapeDtypeStruct((NSLOT, D), jnp.float32),
        mesh=_sc_mesh(),
        scratch_types=[
            pltpu.VMEM((PCH,), jnp.int32),
            pltpu.VMEM((PCH, D), jnp.float32),
            pltpu.SemaphoreType.DMA,
        ],
    )
    def k(table_hbm, slots_hbm, out_hbm, idx_v, rows_v, sem):
        wid = jax.lax.axis_index("s") * 2 + jax.lax.axis_index("c")
        for c in range(PPW // PCH):
            t0 = jax.lax.rem(wid * PPW + c * PCH, S)
            pltpu.sync_copy(table_hbm.at[pl.ds(t0, PCH)], rows_v)
            pltpu.sync_copy(slots_hbm.at[wid, c], idx_v)
            pltpu.async_copy(rows_v, out_hbm.at[idx_v], sem).wait()

    return k(table, slots3)


def _sc_combine(ys, p0, p1, g0, g1, res):
    tok_per_w = S // NSC

    @functools.partial(
        pl.kernel,
        out_type=jax.ShapeDtypeStruct((S, D), jnp.float32),
        mesh=_sc_mesh(),
        scratch_types=[
            pltpu.VMEM((CCH,), jnp.int32),
            pltpu.VMEM((CCH, 16), jnp.float32),
            pltpu.VMEM((CCH, 16), jnp.float32),
            pltpu.VMEM((CCH, D), jnp.float32),
            pltpu.VMEM((CCH, D), jnp.float32),
            pltpu.VMEM((CCH, D), jnp.float32),
            pltpu.SemaphoreType.DMA,
        ],
    )
    def k(ys_hbm, p0_hbm, p1_hbm, g0_hbm, g1_hbm, res_hbm, out_hbm,
          idx_v, g0_v, g1_v, a_v, b_v, r_v, sem):
        wid = jax.lax.axis_index("s") * 2 + jax.lax.axis_index("c")
        base = wid * tok_per_w
        for c in range(tok_per_w // CCH):
            off = base + c * CCH
            pltpu.sync_copy(res_hbm.at[pl.ds(off, CCH)], r_v)
            pltpu.sync_copy(g0_hbm.at[pl.ds(off, CCH)], g0_v)
            pltpu.sync_copy(g1_hbm.at[pl.ds(off, CCH)], g1_v)
            pltpu.sync_copy(p0_hbm.at[pl.ds(off, CCH)], idx_v)
            pltpu.async_copy(ys_hbm.at[idx_v], a_v, sem).wait()
            pltpu.sync_copy(p1_hbm.at[pl.ds(off, CCH)], idx_v)
            pltpu.async_copy(ys_hbm.at[idx_v], b_v, sem).wait()

            def row(rr, carry):
                ga = g0_v[rr, :]
                gb = g1_v[rr, :]
                for cc in range(D // 16):
                    sl = pl.ds(cc * 16, 16)
                    r_v[rr, sl] = (r_v[rr, sl] + ga * a_v[rr, sl]
                                   + gb * b_v[rr, sl])
                return carry

            jax.lax.fori_loop(0, CCH, row, 0)
            pltpu.sync_copy(r_v, out_hbm.at[pl.ds(off, CCH)])

    return k(ys, p0, p1, g0, g1, res)


def _moe(x, gw_t, gb, w1, b1, w2, b2, res):
    slot, val, cnt, rcnt = _gate(x, gw_t, gb)
    p0 = slot[:, 0]
    p1 = slot[:, 1]
    g0 = jnp.broadcast_to(val[:, 0:1], (S, 16))
    g1 = jnp.broadcast_to(val[:, 1:2], (S, 16))
    slots3 = jnp.concatenate([p0, p1]).reshape(NSC, PPW // PCH, PCH)
    nitems = jnp.floor((rcnt + (BT - 1)) * (1.0 / BT))
    cum = jnp.cumsum(nitems)
    e_w = jnp.sum((jnp.arange(NW)[:, None] >= cum[None, :]).astype(
        jnp.int32), axis=1)
    e_w = jnp.clip(e_w, 0, E - 1).astype(jnp.int32)
    xs = _sc_permute(x, slots3)
    ys = _moe_ffn(xs, e_w, w1, b1, w2, b2)
    out = _sc_combine(ys, p0, p1, g0, g1, res)
    aux = E * jnp.sum((cnt / jnp.sum(cnt)) * (cnt / N))
    return out, aux


def kernel(x_context, x_target, in_proj_w, in_proj_b, out_proj_w,
           out_proj_b, gate_w, gate_b, w1, b1, w2, b2, ln_c1_w, ln_c1_b,
           ln_c2_w, ln_c2_b, ln_t1_w, ln_t1_b, ln_t2_w, ln_t2_b):
    xc0 = x_context.reshape(S, D)
    xt0 = x_target.reshape(S, D)
    gw_t = gate_w.T

    xcn = _ln(xc0, ln_c1_w, ln_c1_b)
    xc1 = _mha(xcn, xcn, in_proj_w, in_proj_b, out_proj_w, out_proj_b, xc0)
    xcn2 = _ln(xc1, ln_c2_w, ln_c2_b)
    xc2, aux1 = _moe(xcn2, gw_t, gate_b, w1, b1, w2, b2, xc1)

    xtn = _ln(xt0, ln_t1_w, ln_t1_b)
    xt1 = _mha(xtn, xc2, in_proj_w, in_proj_b, out_proj_w, out_proj_b, xt0)
    xtn2 = _ln(xt1, ln_t2_w, ln_t2_b)
    xt2, aux2 = _moe(xtn2, gw_t, gate_b, w1, b1, w2, b2, xt1)

    aux = 0.01 * (aux1 + aux2)
    return xc2.reshape(S, B, D), xt2.reshape(S, B, D), aux
